# baseline (speedup 1.0000x reference)
import jax
import jax.numpy as jnp
from jax.experimental import pallas as pl

_SEQ_BLK = 1024


def _add_kernel(x_ref, e_ref, o_ref):
    o_ref[...] = x_ref[...] + e_ref[...]


def kernel(inputs, embeddings):
    b, s, d = inputs.shape
    emb = embeddings[:s]
    num_seq = s // _SEQ_BLK
    return pl.pallas_call(
        _add_kernel,
        grid=(num_seq, b),
        in_specs=[
            pl.BlockSpec((1, _SEQ_BLK, d), lambda i, j: (j, i, 0)),
            pl.BlockSpec((_SEQ_BLK, d), lambda i, j: (i, 0)),
        ],
        out_specs=pl.BlockSpec((1, _SEQ_BLK, d), lambda i, j: (j, i, 0)),
        out_shape=jax.ShapeDtypeStruct((b, s, d), inputs.dtype),
    )(inputs, emb)


# speedup vs baseline: 1.0382x; 1.0382x over previous
import jax
import jax.numpy as jnp
from jax.experimental import pallas as pl

_SEQ_BLK = 512


def _add_kernel(x_ref, e_ref, o_ref):
    o_ref[...] = x_ref[...] + e_ref[...]


def kernel(inputs, embeddings):
    b, s, d = inputs.shape
    emb = embeddings[:s]
    num_seq = s // _SEQ_BLK
    return pl.pallas_call(
        _add_kernel,
        grid=(num_seq,),
        in_specs=[
            pl.BlockSpec((b, _SEQ_BLK, d), lambda i: (0, i, 0)),
            pl.BlockSpec((_SEQ_BLK, d), lambda i: (i, 0)),
        ],
        out_specs=pl.BlockSpec((b, _SEQ_BLK, d), lambda i: (0, i, 0)),
        out_shape=jax.ShapeDtypeStruct((b, s, d), inputs.dtype),
    )(inputs, emb)
